# baseline (speedup 1.0000x reference)
import functools

import jax
import jax.numpy as jnp
from jax import lax
from jax.experimental import pallas as pl
from jax.experimental.pallas import tpu as pltpu
from jax.experimental.pallas import tpu_sc as plsc

D = 128
NC = 2
NS = 16
NW = NC * NS

CHUNK = 128
NBUF = 4
SPB = 3


def _make_gather(n_flat):
    b_per_w = n_flat // NW
    n_chunks = b_per_w // CHUNK
    mesh = plsc.VectorSubcoreMesh(core_axis_name="c", subcore_axis_name="s")

    @functools.partial(
        pl.kernel,
        mesh=mesh,
        out_type=jax.ShapeDtypeStruct((n_flat, D), jnp.float32),
        scratch_types=[
            pltpu.VMEM((n_chunks, CHUNK), jnp.int32),
            pltpu.VMEM((NBUF, CHUNK, D), jnp.float32),
            pltpu.VMEM_SHARED((NS, SPB, CHUNK, D), jnp.float32),
            pltpu.SemaphoreType.DMA,
            pltpu.SemaphoreType.DMA,
            pltpu.SemaphoreType.DMA,
        ],
    )
    def gather_kernel(
        idx_hbm, table_hbm, out_hbm, idx_v, rows_v, spm, g_sem, h_sem, d_sem
    ):
        wid = lax.axis_index("s") * NC + lax.axis_index("c")
        sid = lax.axis_index("s")
        base = wid * b_per_w
        pltpu.sync_copy(idx_hbm.at[wid], idx_v)

        gathers = [
            pltpu.async_copy(table_hbm.at[idx_v.at[g]], rows_v.at[g % NBUF], g_sem)
            for g in range(min(NBUF - 1, n_chunks))
        ]
        dmas = []
        for c in range(n_chunks):
            gathers[c].wait()
            if c >= SPB:
                dmas[c - SPB].wait()
            hop = pltpu.async_copy(rows_v.at[c % NBUF], spm.at[sid, c % SPB], h_sem)
            hop.wait()
            g = c + NBUF - 1
            if g < n_chunks:
                gathers.append(
                    pltpu.async_copy(
                        table_hbm.at[idx_v.at[g]], rows_v.at[g % NBUF], g_sem
                    )
                )
            dmas.append(
                pltpu.async_copy(
                    spm.at[sid, c % SPB],
                    out_hbm.at[pl.ds(base + c * CHUNK, CHUNK)],
                    d_sem,
                )
            )
        for c in range(max(0, n_chunks - SPB), n_chunks):
            dmas[c].wait()

    return gather_kernel


def kernel(input_ids, token_embedding):
    b, l = input_ids.shape
    n_flat = b * l
    idx = input_ids.reshape(NW, (n_flat // NW) // CHUNK, CHUNK).astype(jnp.int32)
    out = _make_gather(n_flat)(idx, token_embedding)
    return out.reshape(b, l, D)
